# Initial kernel scaffold; baseline (speedup 1.0000x reference)
import jax
import jax.numpy as jnp
from jax.experimental import pallas as pl


def kernel(x, enc0_w, enc0_s, enc0_t, enc1_w, enc1_s, enc1_t, enc2_w, enc2_s, enc2_t, enc3_w, enc3_s, enc3_t, fcs0_w, fcs0_s, fcs0_t, fcs1_w, fcs1_s, fcs1_t, fcs2_w, fcs2_s, fcs2_t, fcs3_w, fcs3_s, fcs3_t, fcb0_w, fcb0_s, fcb0_t, fcb1_w, fcb1_s, fcb1_t, fcb2_w, fcb2_s, fcb2_t):
    raise NotImplementedError("write your pallas kernel here")


# speedup vs baseline: 1.0115x; 1.0115x over previous
import jax
import jax.numpy as jnp
from jax.experimental import pallas as pl
from jax.experimental.pallas import tpu as pltpu

_SLOPE = 0.01


def _leaky(x):
    return jnp.maximum(x, _SLOPE * x)


_DN = (((0,), (0,)), ((), ()))


def _enc_body(x_ref, w1_ref, t1_ref, w2_ref, t2_ref, w3_ref, t3_ref,
              w4_ref, t4_ref, o_ref):
    x = x_ref[...]
    a = jax.lax.dot_general(w1_ref[...], x, _DN,
                            preferred_element_type=jnp.float32,
                            precision=jax.lax.Precision.HIGHEST)
    a = _leaky(a + t1_ref[...]).astype(jnp.bfloat16)
    a = jax.lax.dot_general(w2_ref[...], a, _DN,
                            preferred_element_type=jnp.float32)
    a = _leaky(a + t2_ref[...]).astype(jnp.bfloat16)
    a = jax.lax.dot_general(w3_ref[...], a, _DN,
                            preferred_element_type=jnp.float32)
    a = _leaky(a + t3_ref[...]).astype(jnp.bfloat16)
    y = jax.lax.dot_general(w4_ref[...], a, _DN,
                            preferred_element_type=jnp.float32)
    o_ref[...] = jnp.max(y, axis=1, keepdims=True) + t4_ref[...]


def _encoder(x, w1, t1, w2, t2, w3, t3, w4, t4):
    B, C, N = x.shape
    cout = w4.shape[1]
    flat = [w1, t1, w2, t2, w3, t3, w4, t4]
    in_specs = [pl.BlockSpec((None, C, N), lambda b: (b, 0, 0))]
    for arr in flat:
        in_specs.append(pl.BlockSpec(arr.shape, lambda b: (0, 0)))
    out = pl.pallas_call(
        _enc_body,
        out_shape=jax.ShapeDtypeStruct((B, cout, 1), jnp.float32),
        grid=(B,),
        in_specs=in_specs,
        out_specs=pl.BlockSpec((None, cout, 1), lambda b: (b, 0, 0)),
        compiler_params=pltpu.CompilerParams(
            dimension_semantics=("parallel",),
            vmem_limit_bytes=32 * 1024 * 1024),
    )(x, *flat)
    return out.reshape(B, cout)


def _head_body(p_ref,
               f0w, f0s, f0t, f1w, f1s, f1t, f2w, f2s, f2t, f3w, f3s, f3t,
               w_ref, s_ref, t_ref, o_ref):
    a = p_ref[...].astype(jnp.bfloat16)
    for wr, sr, tr, has_act in ((f0w, f0s, f0t, True), (f1w, f1s, f1t, True),
                                (f2w, f2s, f2t, False), (f3w, f3s, f3t, True)):
        y = jnp.dot(a, wr[...], preferred_element_type=jnp.float32)
        y = y * sr[...] + tr[...]
        if has_act:
            y = _leaky(y)
        a = y.astype(jnp.bfloat16)
    y = jnp.dot(a, w_ref[...], preferred_element_type=jnp.float32)
    y = _leaky(y * s_ref[...] + t_ref[...])
    o_ref[...] = y.astype(jnp.bfloat16)


def _head_fcb0(pooled, fc_small, w_tiled, s, t):
    n_t, k, tn = w_tiled.shape
    B = pooled.shape[0]
    flat = [pooled]
    in_specs = [pl.BlockSpec(pooled.shape, lambda j: (0, 0))]
    for (fw, fs, ft) in fc_small:
        flat += [fw, fs, ft]
        in_specs += [pl.BlockSpec(fw.shape, lambda j: (0, 0)),
                     pl.BlockSpec(fs.shape, lambda j: (0, 0)),
                     pl.BlockSpec(ft.shape, lambda j: (0, 0))]
    flat += [w_tiled, s, t]
    in_specs += [pl.BlockSpec((None, k, tn), lambda j: (j, 0, 0)),
                 pl.BlockSpec((1, tn), lambda j: (0, j)),
                 pl.BlockSpec((1, tn), lambda j: (0, j))]
    return pl.pallas_call(
        _head_body,
        out_shape=jax.ShapeDtypeStruct((B, n_t * tn), jnp.bfloat16),
        grid=(n_t,),
        in_specs=in_specs,
        out_specs=pl.BlockSpec((B, tn), lambda j: (0, j)),
        compiler_params=pltpu.CompilerParams(
            dimension_semantics=("parallel",),
            vmem_limit_bytes=32 * 1024 * 1024),
    )(*flat)


def _stream_layer(act, w_tiled, s, t, with_act, out_dtype):
    n_t, k, tn = w_tiled.shape
    B = act.shape[0]

    def body(a_ref, w_ref, s_ref, t_ref, o_ref):
        y = jnp.dot(a_ref[...], w_ref[...], preferred_element_type=jnp.float32)
        y = y * s_ref[...] + t_ref[...]
        if with_act:
            y = _leaky(y)
        o_ref[...] = y.astype(out_dtype)

    return pl.pallas_call(
        body,
        out_shape=jax.ShapeDtypeStruct((B, n_t * tn), out_dtype),
        grid=(n_t,),
        in_specs=[pl.BlockSpec((B, k), lambda j: (0, 0)),
                  pl.BlockSpec((None, k, tn), lambda j: (j, 0, 0)),
                  pl.BlockSpec((1, tn), lambda j: (0, j)),
                  pl.BlockSpec((1, tn), lambda j: (0, j))],
        out_specs=pl.BlockSpec((B, tn), lambda j: (0, j)),
        compiler_params=pltpu.CompilerParams(
            dimension_semantics=("parallel",),
            vmem_limit_bytes=32 * 1024 * 1024),
    )(act, w_tiled, s, t)


def kernel(x,
           enc0_w, enc0_s, enc0_t,
           enc1_w, enc1_s, enc1_t,
           enc2_w, enc2_s, enc2_t,
           enc3_w, enc3_s, enc3_t,
           fcs0_w, fcs0_s, fcs0_t,
           fcs1_w, fcs1_s, fcs1_t,
           fcs2_w, fcs2_s, fcs2_t,
           fcs3_w, fcs3_s, fcs3_t,
           fcb0_w, fcb0_s, fcb0_t,
           fcb1_w, fcb1_s, fcb1_t,
           fcb2_w, fcb2_s, fcb2_t):
    f32 = jnp.float32
    bf16 = jnp.bfloat16
    w1 = enc0_w * enc0_s
    w2 = (enc1_w.astype(f32) * enc1_s).astype(bf16)
    w3 = (enc2_w.astype(f32) * enc2_s).astype(bf16)
    w4 = (enc3_w.astype(f32) * enc3_s).astype(bf16)
    t1 = enc0_t.T
    t2 = enc1_t.T
    t3 = enc2_t.T
    t4 = enc3_t.T

    pooled = _encoder(x, w1, t1, w2, t2, w3, t3, w4, t4)

    fc_small = [(fcs0_w, fcs0_s, fcs0_t), (fcs1_w, fcs1_s, fcs1_t),
                (fcs2_w, fcs2_s, fcs2_t), (fcs3_w, fcs3_s, fcs3_t)]
    act1 = _head_fcb0(pooled, fc_small, fcb0_w, fcb0_s, fcb0_t)
    act2 = _stream_layer(act1, fcb1_w, fcb1_s, fcb1_t, True, bf16)
    return _stream_layer(act2, fcb2_w, fcb2_s, fcb2_t, False, f32)


# speedup vs baseline: 1.4623x; 1.4457x over previous
import jax
import jax.numpy as jnp
from jax.experimental import pallas as pl
from jax.experimental.pallas import tpu as pltpu

_SLOPE = 0.01


def _leaky(x):
    return jnp.maximum(x, _SLOPE * x)


_DN = (((0,), (0,)), ((), ()))
_ROWS = 8


def _enc_body(x_ref, w1_ref, t1_ref, w2_ref, t2_ref, w3_ref, t3_ref,
              w4_ref, t4_ref, o_ref):
    cols = []
    for r in range(_ROWS):
        x = x_ref[:, r, :]
        a = jax.lax.dot_general(w1_ref[...], x, _DN,
                                preferred_element_type=jnp.float32)
        a = _leaky(a + t1_ref[...]).astype(jnp.bfloat16)
        a = jax.lax.dot_general(w2_ref[...], a, _DN,
                                preferred_element_type=jnp.float32)
        a = _leaky(a + t2_ref[...]).astype(jnp.bfloat16)
        a = jax.lax.dot_general(w3_ref[...], a, _DN,
                                preferred_element_type=jnp.float32)
        a = _leaky(a + t3_ref[...]).astype(jnp.bfloat16)
        y = jax.lax.dot_general(w4_ref[...], a, _DN,
                                preferred_element_type=jnp.float32)
        cols.append(jnp.max(y, axis=1, keepdims=True))
    p = jnp.concatenate(cols, axis=1)
    o_ref[...] = p.T + t4_ref[...]


def _encoder(xt, w1, t1, w2, t2, w3, t3, w4, t4):
    C, B, N = xt.shape
    cout = w4.shape[1]
    flat = [w1, t1, w2, t2, w3, t3, w4, t4]
    in_specs = [pl.BlockSpec((C, _ROWS, N), lambda g: (0, g, 0))]
    for arr in flat:
        in_specs.append(pl.BlockSpec(arr.shape, lambda g: (0, 0)))
    return pl.pallas_call(
        _enc_body,
        out_shape=jax.ShapeDtypeStruct((B, cout), jnp.float32),
        grid=(B // _ROWS,),
        in_specs=in_specs,
        out_specs=pl.BlockSpec((_ROWS, cout), lambda g: (g, 0)),
        compiler_params=pltpu.CompilerParams(
            dimension_semantics=("parallel",),
            vmem_limit_bytes=32 * 1024 * 1024),
    )(xt, *flat)


def _head_body(p_ref,
               f0w, f0s, f0t, f1w, f1s, f1t, f2w, f2s, f2t, f3w, f3s, f3t,
               w_ref, s_ref, t_ref, o_ref):
    a = p_ref[...].astype(jnp.bfloat16)
    for wr, sr, tr, has_act in ((f0w, f0s, f0t, True), (f1w, f1s, f1t, True),
                                (f2w, f2s, f2t, False), (f3w, f3s, f3t, True)):
        y = jnp.dot(a, wr[...], preferred_element_type=jnp.float32)
        y = y * sr[...] + tr[...]
        if has_act:
            y = _leaky(y)
        a = y.astype(jnp.bfloat16)
    y = jnp.dot(a, w_ref[...], preferred_element_type=jnp.float32)
    y = _leaky(y * s_ref[...] + t_ref[...])
    o_ref[...] = y.astype(jnp.bfloat16)


def _head_fcb0(pooled, fc_small, w_tiled, s, t):
    n_t, k, tn = w_tiled.shape
    B = pooled.shape[0]
    flat = [pooled]
    in_specs = [pl.BlockSpec(pooled.shape, lambda j: (0, 0))]
    for (fw, fs, ft) in fc_small:
        flat += [fw, fs, ft]
        in_specs += [pl.BlockSpec(fw.shape, lambda j: (0, 0)),
                     pl.BlockSpec(fs.shape, lambda j: (0, 0)),
                     pl.BlockSpec(ft.shape, lambda j: (0, 0))]
    flat += [w_tiled, s, t]
    in_specs += [pl.BlockSpec((None, k, tn), lambda j: (j, 0, 0)),
                 pl.BlockSpec((1, tn), lambda j: (0, j)),
                 pl.BlockSpec((1, tn), lambda j: (0, j))]
    return pl.pallas_call(
        _head_body,
        out_shape=jax.ShapeDtypeStruct((B, n_t * tn), jnp.bfloat16),
        grid=(n_t,),
        in_specs=in_specs,
        out_specs=pl.BlockSpec((B, tn), lambda j: (0, j)),
        compiler_params=pltpu.CompilerParams(
            dimension_semantics=("parallel",),
            vmem_limit_bytes=32 * 1024 * 1024),
    )(*flat)


def _stream_layer(act, w_tiled, s, t, with_act, out_dtype):
    n_t, k, tn = w_tiled.shape
    B = act.shape[0]

    def body(a_ref, w_ref, s_ref, t_ref, o_ref):
        y = jnp.dot(a_ref[...], w_ref[...], preferred_element_type=jnp.float32)
        y = y * s_ref[...] + t_ref[...]
        if with_act:
            y = _leaky(y)
        o_ref[...] = y.astype(out_dtype)

    return pl.pallas_call(
        body,
        out_shape=jax.ShapeDtypeStruct((B, n_t * tn), out_dtype),
        grid=(n_t,),
        in_specs=[pl.BlockSpec((B, k), lambda j: (0, 0)),
                  pl.BlockSpec((None, k, tn), lambda j: (j, 0, 0)),
                  pl.BlockSpec((1, tn), lambda j: (0, j)),
                  pl.BlockSpec((1, tn), lambda j: (0, j))],
        out_specs=pl.BlockSpec((B, tn), lambda j: (0, j)),
        compiler_params=pltpu.CompilerParams(
            dimension_semantics=("parallel",),
            vmem_limit_bytes=32 * 1024 * 1024),
    )(act, w_tiled, s, t)


def kernel(x,
           enc0_w, enc0_s, enc0_t,
           enc1_w, enc1_s, enc1_t,
           enc2_w, enc2_s, enc2_t,
           enc3_w, enc3_s, enc3_t,
           fcs0_w, fcs0_s, fcs0_t,
           fcs1_w, fcs1_s, fcs1_t,
           fcs2_w, fcs2_s, fcs2_t,
           fcs3_w, fcs3_s, fcs3_t,
           fcb0_w, fcb0_s, fcb0_t,
           fcb1_w, fcb1_s, fcb1_t,
           fcb2_w, fcb2_s, fcb2_t):
    f32 = jnp.float32
    bf16 = jnp.bfloat16
    w1 = enc0_w * enc0_s
    w2 = (enc1_w.astype(f32) * enc1_s).astype(bf16)
    w3 = (enc2_w.astype(f32) * enc2_s).astype(bf16)
    w4 = (enc3_w.astype(f32) * enc3_s).astype(bf16)
    t1 = enc0_t.T
    t2 = enc1_t.T
    t3 = enc2_t.T

    xt = jnp.transpose(x, (1, 0, 2))
    pooled = _encoder(xt, w1, t1, w2, t2, w3, t3, w4, enc3_t)

    fc_small = [(fcs0_w, fcs0_s, fcs0_t), (fcs1_w, fcs1_s, fcs1_t),
                (fcs2_w, fcs2_s, fcs2_t), (fcs3_w, fcs3_s, fcs3_t)]
    act1 = _head_fcb0(pooled, fc_small, fcb0_w, fcb0_s, fcb0_t)
    act2 = _stream_layer(act1, fcb1_w, fcb1_s, fcb1_t, True, bf16)
    return _stream_layer(act2, fcb2_w, fcb2_s, fcb2_t, False, f32)


# speedup vs baseline: 1.5166x; 1.0371x over previous
import jax
import jax.numpy as jnp
from jax.experimental import pallas as pl
from jax.experimental.pallas import tpu as pltpu

_SLOPE = 0.01


def _leaky(x):
    return jnp.maximum(x, _SLOPE * x)


_DN = (((0,), (0,)), ((), ()))
_ROWS = 8


def _make_enc_body(n, cchunk):
    def _enc_body(x_ref, w1_ref, w2_ref, w3_ref, w4_ref, t4_ref, o_ref):
        s = _ROWS * n
        x = x_ref[...].astype(jnp.bfloat16)
        ones = jnp.ones((1, s), jnp.bfloat16)
        a = jnp.concatenate([x, ones], axis=0)
        for w_ref in (w1_ref, w2_ref, w3_ref):
            a = jax.lax.dot_general(w_ref[...], a, _DN,
                                    preferred_element_type=jnp.float32)
            a = _leaky(a.astype(jnp.bfloat16))
        chunks = []
        cout = w4_ref.shape[1]
        for c0 in range(0, cout, cchunk):
            y = jax.lax.dot_general(w4_ref[:, c0:c0 + cchunk], a, _DN,
                                    preferred_element_type=jnp.float32)
            cols = [jnp.max(y[:, r * n:(r + 1) * n], axis=1, keepdims=True)
                    for r in range(_ROWS)]
            chunks.append(jnp.concatenate(cols, axis=1))
        p = jnp.concatenate(chunks, axis=0)
        o_ref[...] = p.T + t4_ref[...]
    return _enc_body


def _encoder(xt2, n, w1a, w2a, w3a, w4, t4):
    C1, total = xt2.shape
    b = total // n
    cout = w4.shape[1]
    flat = [w1a, w2a, w3a, w4, t4]
    in_specs = [pl.BlockSpec((C1, _ROWS * n), lambda g: (0, g))]
    for arr in flat:
        in_specs.append(pl.BlockSpec(arr.shape, lambda g: (0, 0)))
    return pl.pallas_call(
        _make_enc_body(n, 256),
        out_shape=jax.ShapeDtypeStruct((b, cout), jnp.float32),
        grid=(b // _ROWS,),
        in_specs=in_specs,
        out_specs=pl.BlockSpec((_ROWS, cout), lambda g: (g, 0)),
        compiler_params=pltpu.CompilerParams(
            dimension_semantics=("parallel",),
            vmem_limit_bytes=56 * 1024 * 1024),
    )(xt2, *flat)


def _head_body(p_ref,
               f0w, f0s, f0t, f1w, f1s, f1t, f2w, f2s, f2t, f3w, f3s, f3t,
               w_ref, s_ref, t_ref, o_ref):
    a = p_ref[...].astype(jnp.bfloat16)
    for wr, sr, tr, has_act in ((f0w, f0s, f0t, True), (f1w, f1s, f1t, True),
                                (f2w, f2s, f2t, False), (f3w, f3s, f3t, True)):
        y = jnp.dot(a, wr[...], preferred_element_type=jnp.float32)
        y = y * sr[...] + tr[...]
        if has_act:
            y = _leaky(y)
        a = y.astype(jnp.bfloat16)
    y = jnp.dot(a, w_ref[...], preferred_element_type=jnp.float32)
    y = _leaky(y * s_ref[...] + t_ref[...])
    o_ref[...] = y.astype(jnp.bfloat16)


def _head_fcb0(pooled, fc_small, w_tiled, s, t):
    n_t, k, tn = w_tiled.shape
    B = pooled.shape[0]
    flat = [pooled]
    in_specs = [pl.BlockSpec(pooled.shape, lambda j: (0, 0))]
    for (fw, fs, ft) in fc_small:
        flat += [fw, fs, ft]
        in_specs += [pl.BlockSpec(fw.shape, lambda j: (0, 0)),
                     pl.BlockSpec(fs.shape, lambda j: (0, 0)),
                     pl.BlockSpec(ft.shape, lambda j: (0, 0))]
    flat += [w_tiled, s, t]
    in_specs += [pl.BlockSpec((None, k, tn), lambda j: (j, 0, 0)),
                 pl.BlockSpec((1, tn), lambda j: (0, j)),
                 pl.BlockSpec((1, tn), lambda j: (0, j))]
    return pl.pallas_call(
        _head_body,
        out_shape=jax.ShapeDtypeStruct((B, n_t * tn), jnp.bfloat16),
        grid=(n_t,),
        in_specs=in_specs,
        out_specs=pl.BlockSpec((B, tn), lambda j: (0, j)),
        compiler_params=pltpu.CompilerParams(
            dimension_semantics=("parallel",),
            vmem_limit_bytes=60 * 1024 * 1024),
    )(*flat)


def _stream_layer(act, w_tiled, s, t, with_act, out_dtype):
    n_t, k, tn = w_tiled.shape
    B = act.shape[0]

    def body(a_ref, w_ref, s_ref, t_ref, o_ref):
        y = jnp.dot(a_ref[...], w_ref[...], preferred_element_type=jnp.float32)
        y = y * s_ref[...] + t_ref[...]
        if with_act:
            y = _leaky(y)
        o_ref[...] = y.astype(out_dtype)

    return pl.pallas_call(
        body,
        out_shape=jax.ShapeDtypeStruct((B, n_t * tn), out_dtype),
        grid=(n_t,),
        in_specs=[pl.BlockSpec((B, k), lambda j: (0, 0)),
                  pl.BlockSpec((None, k, tn), lambda j: (j, 0, 0)),
                  pl.BlockSpec((1, tn), lambda j: (0, j)),
                  pl.BlockSpec((1, tn), lambda j: (0, j))],
        out_specs=pl.BlockSpec((B, tn), lambda j: (0, j)),
        compiler_params=pltpu.CompilerParams(
            dimension_semantics=("parallel",),
            vmem_limit_bytes=60 * 1024 * 1024),
    )(act, w_tiled, s, t)


def kernel(x,
           enc0_w, enc0_s, enc0_t,
           enc1_w, enc1_s, enc1_t,
           enc2_w, enc2_s, enc2_t,
           enc3_w, enc3_s, enc3_t,
           fcs0_w, fcs0_s, fcs0_t,
           fcs1_w, fcs1_s, fcs1_t,
           fcs2_w, fcs2_s, fcs2_t,
           fcs3_w, fcs3_s, fcs3_t,
           fcb0_w, fcb0_s, fcb0_t,
           fcb1_w, fcb1_s, fcb1_t,
           fcb2_w, fcb2_s, fcb2_t):
    f32 = jnp.float32
    bf16 = jnp.bfloat16
    def _carrier_col(k):
        return jnp.concatenate(
            [jnp.zeros((k, 1), f32), jnp.ones((1, 1), f32),
             jnp.zeros((7, 1), f32)], axis=0)

    def _aug(wf, t, carrier):
        k, c = wf.shape
        body = jnp.concatenate([wf, t, jnp.zeros((7, c), f32)], axis=0)
        if carrier:
            body = jnp.concatenate(
                [body, _carrier_col(k), jnp.zeros((k + 8, 7), f32)], axis=1)
        return body.astype(bf16)

    c1 = enc0_w.shape[1]
    w1a = jnp.concatenate(
        [jnp.concatenate([enc0_w * enc0_s, enc0_t], axis=0),
         jnp.concatenate([jnp.zeros((3, 1), f32), jnp.ones((1, 1), f32)],
                         axis=0),
         jnp.zeros((4, 7), f32)], axis=1).astype(bf16)
    w2a = _aug(enc1_w.astype(f32) * enc1_s, enc1_t, True)
    w3a = _aug(enc2_w.astype(f32) * enc2_s, enc2_t, False)
    w4 = (enc3_w.astype(f32) * enc3_s).astype(bf16)

    B, C, N = x.shape
    xt2 = jnp.transpose(x, (1, 0, 2)).reshape(C, B * N)
    pooled = _encoder(xt2, N, w1a, w2a, w3a, w4, enc3_t)

    fc_small = [(fcs0_w, fcs0_s, fcs0_t), (fcs1_w, fcs1_s, fcs1_t),
                (fcs2_w, fcs2_s, fcs2_t), (fcs3_w, fcs3_s, fcs3_t)]
    act1 = _head_fcb0(pooled, fc_small, fcb0_w, fcb0_s, fcb0_t)
    act2 = _stream_layer(act1, fcb1_w, fcb1_s, fcb1_t, True, bf16)
    return _stream_layer(act2, fcb2_w, fcb2_s, fcb2_t, False, f32)


# speedup vs baseline: 1.7297x; 1.1405x over previous
import jax
import jax.numpy as jnp
from jax.experimental import pallas as pl
from jax.experimental.pallas import tpu as pltpu

_SLOPE = 0.01


def _leaky(x):
    return jnp.maximum(x, _SLOPE * x)


_DN = (((0,), (0,)), ((), ()))
_ROWS = 8


def _make_enc_body(n, cchunk):
    def _enc_body(x_ref, w1_ref, w2_ref, t2_ref, w3_ref, t3_ref,
                  w4_ref, t4_ref, o_ref):
        s = _ROWS * n
        x = x_ref[...].astype(jnp.bfloat16)
        ones = jnp.ones((1, s), jnp.bfloat16)
        a = jnp.concatenate([x, ones], axis=0)
        for w_ref, t_ref in ((w1_ref, None), (w2_ref, t2_ref),
                             (w3_ref, t3_ref)):
            a = jax.lax.dot_general(w_ref[...], a, _DN,
                                    preferred_element_type=jnp.float32)
            if t_ref is not None:
                a = a + t_ref[...]
            a = _leaky(a.astype(jnp.bfloat16))
        chunks = []
        cout = w4_ref.shape[1]
        for c0 in range(0, cout, cchunk):
            y = jax.lax.dot_general(w4_ref[:, c0:c0 + cchunk], a, _DN,
                                    preferred_element_type=jnp.float32)
            cols = [jnp.max(y[:, r * n:(r + 1) * n], axis=1, keepdims=True)
                    for r in range(_ROWS)]
            chunks.append(jnp.concatenate(cols, axis=1))
        p = jnp.concatenate(chunks, axis=0)
        o_ref[...] = p.T + t4_ref[...]
    return _enc_body


def _encoder(xt2, n, w1a, w2, t2, w3, t3, w4, t4):
    C1, total = xt2.shape
    b = total // n
    cout = w4.shape[1]
    flat = [w1a, w2, t2, w3, t3, w4, t4]
    in_specs = [pl.BlockSpec((C1, _ROWS * n), lambda g: (0, g))]
    for arr in flat:
        in_specs.append(pl.BlockSpec(arr.shape, lambda g: (0, 0)))
    return pl.pallas_call(
        _make_enc_body(n, 256),
        out_shape=jax.ShapeDtypeStruct((b, cout), jnp.float32),
        grid=(b // _ROWS,),
        in_specs=in_specs,
        out_specs=pl.BlockSpec((_ROWS, cout), lambda g: (g, 0)),
        compiler_params=pltpu.CompilerParams(
            dimension_semantics=("parallel",),
            vmem_limit_bytes=56 * 1024 * 1024),
    )(xt2, *flat)


_NSLOT = 4
_AHEAD = 3


def _make_decoder_body(small_acts, big_acts, big_dims, tn):
    n_small = len(small_acts)
    n_big = len(big_acts)
    n_tiles = [n_out // tn for (_, n_out) in big_dims]
    schedule = [(l, j) for l in range(n_big) for j in range(n_tiles[l])]
    total = len(schedule)

    def body(*refs):
        i = 0
        p_ref = refs[i]; i += 1
        small = [refs[i + 3 * k:i + 3 * k + 3] for k in range(n_small)]
        i += 3 * n_small
        big = [refs[i + 3 * k:i + 3 * k + 3] for k in range(n_big)]
        i += 3 * n_big
        o_ref = refs[i]; i += 1
        slots = [[refs[i + _NSLOT * l + u] for u in range(_NSLOT)]
                 for l in range(n_big)]
        i += _NSLOT * n_big
        acts = [refs[i + l] for l in range(n_big - 1)]
        i += n_big - 1
        sem = refs[i]

        def dma(g):
            l, j = schedule[g]
            u = j % _NSLOT
            return pltpu.make_async_copy(big[l][0].at[j], slots[l][u],
                                         sem.at[l, u])

        for g in range(min(_AHEAD, total)):
            dma(g).start()

        a = p_ref[...].astype(jnp.bfloat16)
        for k in range(n_small):
            wr, sr, tr = small[k]
            y = jnp.dot(a, wr[...], preferred_element_type=jnp.float32)
            y = y * sr[...] + tr[...]
            if small_acts[k]:
                y = _leaky(y)
            a = y.astype(jnp.bfloat16)

        cur = a
        for g in range(total):
            l, j = schedule[g]
            _, s_r, t_r = big[l]
            dma(g).wait()
            y = jnp.dot(cur, slots[l][j % _NSLOT][...],
                        preferred_element_type=jnp.float32)
            if g + _AHEAD < total:
                dma(g + _AHEAD).start()
            y = y * s_r[:, j * tn:(j + 1) * tn] + t_r[:, j * tn:(j + 1) * tn]
            if big_acts[l]:
                y = _leaky(y)
            if l + 1 < n_big:
                acts[l][:, j * tn:(j + 1) * tn] = y.astype(jnp.bfloat16)
                if j + 1 == n_tiles[l]:
                    cur = acts[l][...]
            else:
                o_ref[:, j * tn:(j + 1) * tn] = y
    return body


def _decoder(pooled, fc_small, fc_big):
    B = pooled.shape[0]
    tn = fc_big[0][0].shape[2]
    big_dims = [(w.shape[1], w.shape[0] * w.shape[2]) for (w, _, _) in fc_big]
    n_out = big_dims[-1][1]

    flat = [pooled]
    in_specs = [pl.BlockSpec(pooled.shape, lambda i: (0, 0))]
    for (w, s, t) in fc_small:
        flat += [w, s, t]
        in_specs += [pl.BlockSpec(w.shape, lambda i: (0, 0)),
                     pl.BlockSpec(s.shape, lambda i: (0, 0)),
                     pl.BlockSpec(t.shape, lambda i: (0, 0))]
    for (w, s, t) in fc_big:
        flat += [w, s, t]
        in_specs += [pl.BlockSpec(memory_space=pl.ANY),
                     pl.BlockSpec(s.shape, lambda i: (0, 0)),
                     pl.BlockSpec(t.shape, lambda i: (0, 0))]

    scratch_shapes = []
    for (k_in, _) in big_dims:
        for _u in range(_NSLOT):
            scratch_shapes.append(pltpu.VMEM((k_in, tn), jnp.bfloat16))
    for (_, n_mid) in big_dims[:-1]:
        scratch_shapes.append(pltpu.VMEM((B, n_mid), jnp.bfloat16))
    scratch_shapes.append(pltpu.SemaphoreType.DMA((len(big_dims), _NSLOT)))

    return pl.pallas_call(
        _make_decoder_body((True, True, False, True), (True, True, False),
                           big_dims, tn),
        out_shape=jax.ShapeDtypeStruct((B, n_out), jnp.float32),
        grid=(1,),
        in_specs=in_specs,
        out_specs=pl.BlockSpec((B, n_out), lambda i: (0, 0)),
        scratch_shapes=scratch_shapes,
        compiler_params=pltpu.CompilerParams(
            dimension_semantics=("arbitrary",),
            vmem_limit_bytes=48 * 1024 * 1024),
    )(*flat)


def kernel(x,
           enc0_w, enc0_s, enc0_t,
           enc1_w, enc1_s, enc1_t,
           enc2_w, enc2_s, enc2_t,
           enc3_w, enc3_s, enc3_t,
           fcs0_w, fcs0_s, fcs0_t,
           fcs1_w, fcs1_s, fcs1_t,
           fcs2_w, fcs2_s, fcs2_t,
           fcs3_w, fcs3_s, fcs3_t,
           fcb0_w, fcb0_s, fcb0_t,
           fcb1_w, fcb1_s, fcb1_t,
           fcb2_w, fcb2_s, fcb2_t):
    f32 = jnp.float32
    bf16 = jnp.bfloat16
    w1a = jnp.concatenate([enc0_w * enc0_s, enc0_t], axis=0).astype(bf16)
    w2 = (enc1_w.astype(f32) * enc1_s).astype(bf16)
    w3 = (enc2_w.astype(f32) * enc2_s).astype(bf16)
    w4 = (enc3_w.astype(f32) * enc3_s).astype(bf16)
    t2 = enc1_t.T
    t3 = enc2_t.T

    B, C, N = x.shape
    xt2 = jnp.transpose(x, (1, 0, 2)).reshape(C, B * N)
    pooled = _encoder(xt2, N, w1a, w2, t2, w3, t3, w4, enc3_t)

    fc_small = [(fcs0_w, fcs0_s, fcs0_t), (fcs1_w, fcs1_s, fcs1_t),
                (fcs2_w, fcs2_s, fcs2_t), (fcs3_w, fcs3_s, fcs3_t)]
    fc_big = [(fcb0_w, fcb0_s, fcb0_t), (fcb1_w, fcb1_s, fcb1_t),
              (fcb2_w, fcb2_s, fcb2_t)]
    return _decoder(pooled, fc_small, fc_big)


# speedup vs baseline: 1.8873x; 1.0911x over previous
import jax
import jax.numpy as jnp
from jax.experimental import pallas as pl
from jax.experimental.pallas import tpu as pltpu

_SLOPE = 0.01


def _leaky(x):
    return jnp.maximum(x, _SLOPE * x)


_DN = (((0,), (0,)), ((), ()))
_ROWS = 8


def _make_enc_body(n, cchunk):
    def _enc_body(x_ref, w1_ref, s1_ref, t1_ref, w2_ref, s2_ref, t2_ref,
                  w3_ref, s3_ref, t3_ref, w4_ref, s4_ref, t4_ref, o_ref):
        f32 = jnp.float32
        bf16 = jnp.bfloat16
        w1a = jnp.concatenate([w1_ref[...] * s1_ref[...], t1_ref[...]],
                              axis=0).astype(bf16)
        w2 = (w2_ref[...].astype(f32) * s2_ref[...]).astype(bf16)
        w3 = (w3_ref[...].astype(f32) * s3_ref[...]).astype(bf16)
        w4 = (w4_ref[...].astype(f32) * s4_ref[...]).astype(bf16)
        t2 = t2_ref[...].T
        t3 = t3_ref[...].T

        s = _ROWS * n
        x = x_ref[...].astype(bf16)
        ones = jnp.ones((1, s), bf16)
        a = jnp.concatenate([x, ones], axis=0)
        for w, t in ((w1a, None), (w2, t2), (w3, t3)):
            a = jax.lax.dot_general(w, a, _DN,
                                    preferred_element_type=f32)
            if t is not None:
                a = a + t
            a = _leaky(a.astype(bf16))
        chunks = []
        cout = w4.shape[1]
        for c0 in range(0, cout, cchunk):
            y = jax.lax.dot_general(w4[:, c0:c0 + cchunk], a, _DN,
                                    preferred_element_type=f32)
            cols = [jnp.max(y[:, r * n:(r + 1) * n], axis=1, keepdims=True)
                    for r in range(_ROWS)]
            chunks.append(jnp.concatenate(cols, axis=1))
        p = jnp.concatenate(chunks, axis=0)
        o_ref[...] = p.T + t4_ref[...]
    return _enc_body


def _encoder(xt2, n, enc_params):
    C, total = xt2.shape
    b = total // n
    cout = enc_params[-1][0].shape[1]
    flat = []
    for (w, s, t) in enc_params:
        flat += [w, s, t]
    in_specs = [pl.BlockSpec((C, _ROWS * n), lambda g: (0, g))]
    for arr in flat:
        in_specs.append(pl.BlockSpec(arr.shape, lambda g: (0, 0)))
    return pl.pallas_call(
        _make_enc_body(n, 256),
        out_shape=jax.ShapeDtypeStruct((b, cout), jnp.float32),
        grid=(b // _ROWS,),
        in_specs=in_specs,
        out_specs=pl.BlockSpec((_ROWS, cout), lambda g: (g, 0)),
        compiler_params=pltpu.CompilerParams(
            dimension_semantics=("parallel",),
            vmem_limit_bytes=56 * 1024 * 1024),
    )(xt2, *flat)


_NSLOT = 4
_AHEAD = 3


def _make_decoder_body(small_acts, big_acts, big_dims, tn):
    n_small = len(small_acts)
    n_big = len(big_acts)
    n_tiles = [n_out // tn for (_, n_out) in big_dims]
    schedule = [(l, j) for l in range(n_big) for j in range(n_tiles[l])]
    total = len(schedule)

    def body(*refs):
        i = 0
        p_ref = refs[i]; i += 1
        small = [refs[i + 3 * k:i + 3 * k + 3] for k in range(n_small)]
        i += 3 * n_small
        big = [refs[i + 3 * k:i + 3 * k + 3] for k in range(n_big)]
        i += 3 * n_big
        o_ref = refs[i]; i += 1
        slots = [[refs[i + _NSLOT * l + u] for u in range(_NSLOT)]
                 for l in range(n_big)]
        i += _NSLOT * n_big
        acts = [refs[i + l] for l in range(n_big - 1)]
        i += n_big - 1
        sem = refs[i]

        def dma(g):
            l, j = schedule[g]
            u = j % _NSLOT
            return pltpu.make_async_copy(big[l][0].at[j], slots[l][u],
                                         sem.at[l, u])

        for g in range(min(_AHEAD, total)):
            dma(g).start()

        a = p_ref[...].astype(jnp.bfloat16)
        for k in range(n_small):
            wr, sr, tr = small[k]
            y = jnp.dot(a, wr[...], preferred_element_type=jnp.float32)
            y = y * sr[...] + tr[...]
            if small_acts[k]:
                y = _leaky(y)
            a = y.astype(jnp.bfloat16)

        cur = a
        for g in range(total):
            l, j = schedule[g]
            _, s_r, t_r = big[l]
            dma(g).wait()
            y = jnp.dot(cur, slots[l][j % _NSLOT][...],
                        preferred_element_type=jnp.float32)
            if g + _AHEAD < total:
                dma(g + _AHEAD).start()
            y = y * s_r[:, j * tn:(j + 1) * tn] + t_r[:, j * tn:(j + 1) * tn]
            if big_acts[l]:
                y = _leaky(y)
            if l + 1 < n_big:
                acts[l][:, j * tn:(j + 1) * tn] = y.astype(jnp.bfloat16)
                if j + 1 == n_tiles[l]:
                    cur = acts[l][...]
            else:
                o_ref[:, j * tn:(j + 1) * tn] = y
    return body


def _decoder(pooled, fc_small, fc_big):
    B = pooled.shape[0]
    tn = fc_big[0][0].shape[2]
    big_dims = [(w.shape[1], w.shape[0] * w.shape[2]) for (w, _, _) in fc_big]
    n_out = big_dims[-1][1]

    flat = [pooled]
    in_specs = [pl.BlockSpec(pooled.shape, lambda i: (0, 0))]
    for (w, s, t) in fc_small:
        flat += [w, s, t]
        in_specs += [pl.BlockSpec(w.shape, lambda i: (0, 0)),
                     pl.BlockSpec(s.shape, lambda i: (0, 0)),
                     pl.BlockSpec(t.shape, lambda i: (0, 0))]
    for (w, s, t) in fc_big:
        flat += [w, s, t]
        in_specs += [pl.BlockSpec(memory_space=pl.ANY),
                     pl.BlockSpec(s.shape, lambda i: (0, 0)),
                     pl.BlockSpec(t.shape, lambda i: (0, 0))]

    scratch_shapes = []
    for (k_in, _) in big_dims:
        for _u in range(_NSLOT):
            scratch_shapes.append(pltpu.VMEM((k_in, tn), jnp.bfloat16))
    for (_, n_mid) in big_dims[:-1]:
        scratch_shapes.append(pltpu.VMEM((B, n_mid), jnp.bfloat16))
    scratch_shapes.append(pltpu.SemaphoreType.DMA((len(big_dims), _NSLOT)))

    return pl.pallas_call(
        _make_decoder_body((True, True, False, True), (True, True, False),
                           big_dims, tn),
        out_shape=jax.ShapeDtypeStruct((B, n_out), jnp.float32),
        grid=(1,),
        in_specs=in_specs,
        out_specs=pl.BlockSpec((B, n_out), lambda i: (0, 0)),
        scratch_shapes=scratch_shapes,
        compiler_params=pltpu.CompilerParams(
            dimension_semantics=("arbitrary",),
            vmem_limit_bytes=48 * 1024 * 1024),
    )(*flat)


def kernel(x,
           enc0_w, enc0_s, enc0_t,
           enc1_w, enc1_s, enc1_t,
           enc2_w, enc2_s, enc2_t,
           enc3_w, enc3_s, enc3_t,
           fcs0_w, fcs0_s, fcs0_t,
           fcs1_w, fcs1_s, fcs1_t,
           fcs2_w, fcs2_s, fcs2_t,
           fcs3_w, fcs3_s, fcs3_t,
           fcb0_w, fcb0_s, fcb0_t,
           fcb1_w, fcb1_s, fcb1_t,
           fcb2_w, fcb2_s, fcb2_t):
    f32 = jnp.float32
    bf16 = jnp.bfloat16
    B, C, N = x.shape
    xt2 = jnp.transpose(x, (1, 0, 2)).reshape(C, B * N)
    enc_params = [(enc0_w, enc0_s, enc0_t), (enc1_w, enc1_s, enc1_t),
                  (enc2_w, enc2_s, enc2_t), (enc3_w, enc3_s, enc3_t)]
    pooled = _encoder(xt2, N, enc_params)

    fc_small = [(fcs0_w, fcs0_s, fcs0_t), (fcs1_w, fcs1_s, fcs1_t),
                (fcs2_w, fcs2_s, fcs2_t), (fcs3_w, fcs3_s, fcs3_t)]
    fc_big = [(fcb0_w, fcb0_s, fcb0_t), (fcb1_w, fcb1_s, fcb1_t),
              (fcb2_w, fcb2_s, fcb2_t)]
    return _decoder(pooled, fc_small, fc_big)
